# Initial kernel scaffold; baseline (speedup 1.0000x reference)
import jax
import jax.numpy as jnp
from jax.experimental import pallas as pl


def kernel(x, expert_idx, W1, b1, W2, b2):
    raise NotImplementedError("write your pallas kernel here")


# speedup vs baseline: 8.1683x; 8.1683x over previous
import functools

import jax
import jax.numpy as jnp
from jax import lax
from jax.experimental import pallas as pl
from jax.experimental.pallas import tpu as pltpu
from jax.experimental.pallas import tpu_sc as plsc

E = 64
D = 768
F = 3072
T = 4096

BM = 128
NB = T // BM + E
P = NB * BM

_NTILES = 32
_CHUNK = T // _NTILES


def _wid():
    return lax.axis_index("s") * 2 + lax.axis_index("c")


def _hist_body(eidx_hbm, cnts_hbm, lrank_hbm,
               eidx_v, lrank_v, counts_v, tmpA, tmpB):
    wid = _wid()
    base = wid * _CHUNK
    lane = lax.iota(jnp.int32, 16)
    zeros16 = jnp.zeros((16,), jnp.int32)

    pltpu.sync_copy(eidx_hbm.at[pl.ds(base, _CHUNK)], eidx_v)

    for c in range(E // 16):
        counts_v[pl.ds(c * 16, 16)] = zeros16

    for j in range(_CHUNK // 16):
        ev = eidx_v[pl.ds(j * 16, 16)]
        cnt_before = plsc.load_gather(counts_v, [ev])
        sk, sv = plsc.sort_key_val(ev, lane)
        tmpA[...] = sk
        prev = plsc.load_gather(tmpA, [jnp.maximum(lane - 1, 0)])
        is_start = jnp.logical_or(sk != prev, lane == 0)
        seg_start = plsc.cummax(jnp.where(is_start, lane, zeros16))
        rank_sorted = lane - seg_start
        plsc.store_scatter(tmpA, [sv], rank_sorted)
        lrank_v[pl.ds(j * 16, 16)] = cnt_before + tmpA[...]
        tmpB[...] = cnt_before
        cnt_before_s = plsc.load_gather(tmpB, [sv])
        tmpA[...] = is_start.astype(jnp.int32)
        nxt = plsc.load_gather(tmpA, [jnp.minimum(lane + 1, 15)])
        is_last = jnp.logical_or(nxt == 1, lane == 15)
        plsc.store_scatter(counts_v, [sk], cnt_before_s + rank_sorted + 1,
                           mask=is_last)

    pltpu.sync_copy(counts_v, cnts_hbm.at[wid])
    pltpu.sync_copy(lrank_v, lrank_hbm.at[pl.ds(base, _CHUNK)])


_hist = functools.partial(
    pl.kernel,
    out_type=(
        jax.ShapeDtypeStruct((_NTILES, E), jnp.int32),
        jax.ShapeDtypeStruct((T,), jnp.int32),
    ),
    mesh=plsc.VectorSubcoreMesh(core_axis_name="c", subcore_axis_name="s",
                                num_cores=2, num_subcores=16),
    compiler_params=pltpu.CompilerParams(needs_layout_passes=False),
    scratch_types=(
        pltpu.VMEM((_CHUNK,), jnp.int32),
        pltpu.VMEM((_CHUNK,), jnp.int32),
        pltpu.VMEM((E,), jnp.int32),
        pltpu.VMEM((16,), jnp.int32),
        pltpu.VMEM((16,), jnp.int32),
    ),
)(_hist_body)


def _route_body(eidx_hbm, cnts_hbm, lrank_hbm,
                pos_hbm, bexp_hbm, nbinfo_hbm,
                eidx_v, lrank_v, tmpA, tmpB, cb_v,
                posf_v, m_v, grid_v):
    wid = _wid()
    base = wid * _CHUNK
    lane = lax.iota(jnp.int32, 16)
    zeros16 = jnp.zeros((16,), jnp.int32)
    fifteen = jnp.full((16,), 15, jnp.int32)

    pltpu.sync_copy(eidx_hbm.at[pl.ds(base, _CHUNK)], eidx_v)
    pltpu.sync_copy(lrank_hbm.at[pl.ds(base, _CHUNK)], lrank_v)
    pltpu.sync_copy(cnts_hbm, grid_v)

    wid_vec = lax.broadcast_in_dim(wid, (16,), ())
    tot = []
    pri = []
    for c in range(E // 16):
        t_acc = zeros16
        p_acc = zeros16
        for w in range(_NTILES):
            row = grid_v[w, pl.ds(c * 16, 16)]
            t_acc = t_acc + row
            sel = jnp.full((16,), w, jnp.int32) < wid_vec
            p_acc = p_acc + jnp.where(sel, row, zeros16)
        tot.append(t_acc)
        pri.append(p_acc)

    carry = zeros16
    astart_blk = []
    nbl = []
    for c in range(E // 16):
        nb_c = (tot[c] + (BM - 1)) >> 7
        cs = plsc.cumsum(nb_c)
        astart_blk.append(carry + cs - nb_c)
        tmpA[...] = cs + carry
        carry = plsc.load_gather(tmpA, [fifteen])
        nbl.append(nb_c)

    for c in range(E // 16):
        cb_v[pl.ds(c * 16, 16)] = (astart_blk[c] << 7) + pri[c]

    for j in range(_CHUNK // 16):
        ev = eidx_v[pl.ds(j * 16, 16)]
        cbase = plsc.load_gather(cb_v, [ev])
        pv = cbase + lrank_v[pl.ds(j * 16, 16)]
        posf_v[pl.ds(j * 16, 16)] = pv
    pltpu.sync_copy(posf_v, pos_hbm.at[pl.ds(base, _CHUNK)])

    @pl.when(wid == 0)
    def _():
        for c in range(NB // 16):
            m_v[pl.ds(c * 16, 16)] = zeros16
        for c in range(E // 16):
            evals = jnp.full((16,), c * 16, jnp.int32) + lane
            plsc.store_scatter(m_v, [astart_blk[c]], evals, mask=nbl[c] > 0)
        carry2 = zeros16
        for c in range(NB // 16):
            cm = plsc.cummax(jnp.maximum(m_v[pl.ds(c * 16, 16)], carry2))
            m_v[pl.ds(c * 16, 16)] = cm
            tmpA[...] = cm
            carry2 = plsc.load_gather(tmpA, [fifteen])
        pltpu.sync_copy(m_v, bexp_hbm)
        s = zeros16
        for c in range(E // 16):
            s = s + nbl[c]
        tmpA[...] = plsc.cumsum(s)
        nbu = plsc.load_gather(tmpA, [fifteen])
        tmpB[...] = nbu
        pltpu.sync_copy(tmpB, nbinfo_hbm)


_route = functools.partial(
    pl.kernel,
    out_type=(
        jax.ShapeDtypeStruct((T,), jnp.int32),
        jax.ShapeDtypeStruct((NB,), jnp.int32),
        jax.ShapeDtypeStruct((16,), jnp.int32),
    ),
    mesh=plsc.VectorSubcoreMesh(core_axis_name="c", subcore_axis_name="s",
                                num_cores=2, num_subcores=16),
    compiler_params=pltpu.CompilerParams(needs_layout_passes=False),
    scratch_types=(
        pltpu.VMEM((_CHUNK,), jnp.int32),
        pltpu.VMEM((_CHUNK,), jnp.int32),
        pltpu.VMEM((16,), jnp.int32),
        pltpu.VMEM((16,), jnp.int32),
        pltpu.VMEM((E,), jnp.int32),
        pltpu.VMEM((_CHUNK,), jnp.int32),
        pltpu.VMEM((NB,), jnp.int32),
        pltpu.VMEM((_NTILES, E), jnp.int32),
    ),
)(_route_body)


_PIECE = 64
_NPIECE = P // _PIECE
_PIECE_PER_TILE = _NPIECE // _NTILES


def _dispatch_body(x_hbm, pos_hbm, nbinfo_hbm,
                   xpad_hbm, pos_all_v, inv_v, nbinfo_v, xrows_v, sem):
    wid = _wid()
    lane = lax.iota(jnp.int32, 16)
    zeros16 = jnp.zeros((16,), jnp.int32)

    pltpu.sync_copy(pos_hbm, pos_all_v)
    pltpu.sync_copy(nbinfo_hbm, nbinfo_v)
    nbu = nbinfo_v[...][0]

    for k in range(_PIECE_PER_TILE):
        pbase = (k * _NTILES + wid) * _PIECE
        for v in range(_PIECE // 16):
            inv_v[pl.ds(pbase + v * 16, 16)] = zeros16

    for j in range(T // 16):
        pv = pos_all_v[pl.ds(j * 16, 16)]
        tok = jnp.full((16,), j * 16, jnp.int32) + lane
        plsc.store_scatter(inv_v, [pv], tok)

    for k in range(_PIECE_PER_TILE):
        p = k * _NTILES + wid
        pbase = p * _PIECE

        @pl.when(pbase < nbu * BM)
        def _():
            pltpu.async_copy(x_hbm.at[inv_v.at[pl.ds(pbase, _PIECE)]],
                             xrows_v, sem).wait()
            pltpu.sync_copy(xrows_v, xpad_hbm.at[pl.ds(pbase, _PIECE)])


_dispatch = functools.partial(
    pl.kernel,
    out_type=jax.ShapeDtypeStruct((P, D), jnp.float32),
    mesh=plsc.VectorSubcoreMesh(core_axis_name="c", subcore_axis_name="s",
                                num_cores=2, num_subcores=16),
    compiler_params=pltpu.CompilerParams(needs_layout_passes=False),
    scratch_types=(
        pltpu.VMEM((T,), jnp.int32),
        pltpu.VMEM((P,), jnp.int32),
        pltpu.VMEM((16,), jnp.int32),
        pltpu.VMEM((_PIECE, D), jnp.float32),
        pltpu.SemaphoreType.DMA,
    ),
)(_dispatch_body)


def _gemm_body(nbinfo_ref, bexp_ref, x_ref, w1_ref, b1_ref, w2_ref, b2_ref,
               out_ref):
    g = pl.program_id(0)

    @pl.when(g < nbinfo_ref[0])
    def _():
        h = jnp.dot(x_ref[...], w1_ref[0], preferred_element_type=jnp.float32)
        h = jax.nn.gelu(h + b1_ref[0])
        y = jnp.dot(h, w2_ref[0], preferred_element_type=jnp.float32)
        out_ref[...] = y + b2_ref[0]


def _gemm(nbinfo, bexp, x_padded, W1, b1, W2, b2):
    def _g(g, nbinfo, bexp):
        return jnp.minimum(g, nbinfo[0] - 1)

    def xmap(g, nbinfo, bexp):
        return (_g(g, nbinfo, bexp), 0)

    def wmap(g, nbinfo, bexp):
        return (bexp[_g(g, nbinfo, bexp)], 0, 0)

    def bmap(g, nbinfo, bexp):
        return (bexp[_g(g, nbinfo, bexp)], 0, 0)

    grid_spec = pltpu.PrefetchScalarGridSpec(
        num_scalar_prefetch=2,
        grid=(NB,),
        in_specs=[
            pl.BlockSpec((BM, D), xmap),
            pl.BlockSpec((1, D, F), wmap),
            pl.BlockSpec((1, 1, F), bmap),
            pl.BlockSpec((1, F, D), wmap),
            pl.BlockSpec((1, 1, D), bmap),
        ],
        out_specs=pl.BlockSpec((BM, D), xmap),
    )
    return pl.pallas_call(
        _gemm_body,
        grid_spec=grid_spec,
        out_shape=jax.ShapeDtypeStruct((P, D), jnp.float32),
        compiler_params=pltpu.CompilerParams(
            dimension_semantics=("arbitrary",)),
    )(nbinfo, bexp, x_padded, W1, b1.reshape(E, 1, F), W2,
      b2.reshape(E, 1, D))


def _combine_body(ypad_hbm, pos_hbm, eidx_hbm, b2_hbm,
                  out_hbm, bias_hbm, idx_v, rows_v, sem):
    wid = _wid()
    base = wid * _CHUNK
    pltpu.sync_copy(pos_hbm.at[pl.ds(base, _CHUNK)], idx_v)
    pltpu.async_copy(ypad_hbm.at[idx_v], rows_v, sem).wait()
    pltpu.sync_copy(rows_v, out_hbm.at[pl.ds(base, _CHUNK)])
    pltpu.sync_copy(eidx_hbm.at[pl.ds(base, _CHUNK)], idx_v)
    pltpu.async_copy(b2_hbm.at[idx_v], rows_v, sem).wait()
    pltpu.sync_copy(rows_v, bias_hbm.at[pl.ds(base, _CHUNK)])


_combine = functools.partial(
    pl.kernel,
    out_type=(
        jax.ShapeDtypeStruct((T, D), jnp.float32),
        jax.ShapeDtypeStruct((T, D), jnp.float32),
    ),
    mesh=plsc.VectorSubcoreMesh(core_axis_name="c", subcore_axis_name="s",
                                num_cores=2, num_subcores=16),
    compiler_params=pltpu.CompilerParams(needs_layout_passes=False),
    scratch_types=(
        pltpu.VMEM((_CHUNK,), jnp.int32),
        pltpu.VMEM((_CHUNK, D), jnp.float32),
        pltpu.SemaphoreType.DMA,
    ),
)(_combine_body)


def kernel(x, expert_idx, W1, b1, W2, b2):
    eidx = expert_idx.astype(jnp.int32)
    cnts, lrank = _hist(eidx)
    pos, bexp, nbinfo = _route(eidx, cnts, lrank)
    x_padded = _dispatch(x, pos, nbinfo)
    y_padded = _gemm(nbinfo, bexp, x_padded, W1, b1, W2, b2)
    out, mlp_bias = _combine(y_padded, pos, eidx, b2)
    return (out, mlp_bias)
